# Initial kernel scaffold; baseline (speedup 1.0000x reference)
import jax
import jax.numpy as jnp
from jax.experimental import pallas as pl


def kernel(num_point, f0, f1, f2, f3, f4, FPS_0, FPS_1, FPS_2, FPS_3, W04, b04, W14, b14, W24, b24, W34, b34, W4, b4):
    raise NotImplementedError("write your pallas kernel here")


# speedup vs baseline: 4.0325x; 4.0325x over previous
import functools

import jax
import jax.numpy as jnp
from jax import lax
from jax.experimental import pallas as pl
from jax.experimental.pallas import tpu as pltpu
from jax.experimental.pallas import tpu_sc as plsc

_B = 128
_S = 128
_N0, _N1, _N2, _N3 = 2048, 1024, 512, 256
_C0, _C1, _C2, _C3, _C4 = 64, 128, 128, 256, 256
_M = _B * _S
_BLK = 2048
_EPS = 1e-5
_NC, _NS = 2, 16
_NW = _NC * _NS
_BPW = _B // _NW


def _sc_gather():
    mesh = plsc.VectorSubcoreMesh(
        core_axis_name="c", subcore_axis_name="s",
        num_cores=_NC, num_subcores=_NS)

    @functools.partial(
        pl.kernel,
        mesh=mesh,
        compiler_params=pltpu.CompilerParams(
            needs_layout_passes=False, use_tc_tiling_on_sc=False),
        out_type=(
            jax.ShapeDtypeStruct((_M, _C0), jnp.float32),
            jax.ShapeDtypeStruct((_M, _C1), jnp.float32),
            jax.ShapeDtypeStruct((_M, _C2), jnp.float32),
            jax.ShapeDtypeStruct((_M, _C3), jnp.float32),
        ),
        scratch_types=[
            pltpu.VMEM((_N1,), jnp.int32),
            pltpu.VMEM((_N2,), jnp.int32),
            pltpu.VMEM((_N3,), jnp.int32),
            pltpu.VMEM((_S,), jnp.int32),
            pltpu.VMEM((_S,), jnp.int32),
            pltpu.VMEM((_S,), jnp.int32),
            pltpu.VMEM((_S,), jnp.int32),
            pltpu.VMEM((_S,), jnp.int32),
            pltpu.VMEM((_S, _C0), jnp.float32),
            pltpu.VMEM((_S, _C1), jnp.float32),
            pltpu.VMEM((_S, _C2), jnp.float32),
            pltpu.VMEM((_S, _C3), jnp.float32),
            pltpu.SemaphoreType.DMA,
        ],
    )
    def body(fps0, fps1, fps2, fps3, f0, f1, f2, f3,
             g0, g1, g2, g3,
             t0, t1, t2, t3, gi0, gi1, gi2, gi3, r0, r1, r2, r3, sem):
        wid = lax.axis_index("s") * _NC + lax.axis_index("c")

        def per_batch(j, carry):
            b = wid * _BPW + j
            pltpu.sync_copy(fps0.at[b], t0)
            pltpu.sync_copy(fps1.at[b], t1)
            pltpu.sync_copy(fps2.at[b], t2)
            pltpu.sync_copy(fps3.at[b], t3)

            def chain(i, c):
                s = pl.ds(i * 16, 16)
                v3 = t3[s]
                v2 = plsc.load_gather(t2, [v3])
                v1 = plsc.load_gather(t1, [v2])
                v0 = plsc.load_gather(t0, [v1])
                gi3[s] = v3 + b * _N3
                gi2[s] = v2 + b * _N2
                gi1[s] = v1 + b * _N1
                gi0[s] = v0 + b * _N0
                return c

            lax.fori_loop(0, _S // 16, chain, jnp.int32(0))

            c0 = pltpu.async_copy(f0.at[gi0], r0, sem)
            c1 = pltpu.async_copy(f1.at[gi1], r1, sem)
            c2 = pltpu.async_copy(f2.at[gi2], r2, sem)
            c3 = pltpu.async_copy(f3.at[gi3], r3, sem)
            c0.wait()
            c1.wait()
            c2.wait()
            c3.wait()
            row = b * _S
            pltpu.sync_copy(r0, g0.at[pl.ds(row, _S)])
            pltpu.sync_copy(r1, g1.at[pl.ds(row, _S)])
            pltpu.sync_copy(r2, g2.at[pl.ds(row, _S)])
            pltpu.sync_copy(r3, g3.at[pl.ds(row, _S)])
            return carry

        lax.fori_loop(0, _BPW, per_batch, jnp.int32(0))

    return body


def _mm(x, w):
    return lax.dot_general(x, w, (((1,), (1,)), ((), ())),
                           preferred_element_type=jnp.float32)


def _stats_body(g0, g1, g2, g3, w04, w14, w24, w34,
                b04, b14, b24, b34, st):
    i = pl.program_id(0)

    @pl.when(i == 0)
    def _init():
        st[...] = jnp.zeros_like(st)

    for k, (g, w, bb) in enumerate(((g0, w04, b04), (g1, w14, b14),
                                    (g2, w24, b24), (g3, w34, b34))):
        z = _mm(g[...], w[...]) + bb[...]
        st[k:k + 1, :] += jnp.sum(z, axis=0, keepdims=True)
        st[k + 4:k + 5, :] += jnp.sum(z * z, axis=0, keepdims=True)


def _mid_body(st, g0, g1, g2, g3, f4b, w04, w14, w24, w34,
              b04, b14, b24, b34, w4, b4, z5, st5):
    i = pl.program_id(0)

    @pl.when(i == 0)
    def _init():
        st5[...] = jnp.zeros_like(st5)

    stv = st[...]
    acc = f4b[...]
    for k, (g, w, bb) in enumerate(((g0, w04, b04), (g1, w14, b14),
                                    (g2, w24, b24), (g3, w34, b34))):
        z = _mm(g[...], w[...]) + bb[...]
        mu = stv[k:k + 1, :] * (1.0 / _M)
        ex2 = stv[k + 4:k + 5, :] * (1.0 / _M)
        rs = lax.rsqrt(ex2 - mu * mu + _EPS)
        y = (z - mu) * rs
        acc = acc + jnp.where(y > 0, y, 0.2 * y)
    z = _mm(acc, w4[...]) + b4[...]
    z5[...] = z
    st5[0:1, :] += jnp.sum(z, axis=0, keepdims=True)
    st5[1:2, :] += jnp.sum(z * z, axis=0, keepdims=True)


def _fin_body(st5, z5b, f4b, ob):
    stv = st5[...]
    mu = stv[0:1, :] * (1.0 / _M)
    ex2 = stv[1:2, :] * (1.0 / _M)
    rs = lax.rsqrt(ex2 - mu * mu + _EPS)
    y = (z5b[...] - mu) * rs
    ob[...] = jnp.where(y > 0, y, 0.2 * y) + f4b[...]


def _rows(c):
    return pl.BlockSpec((_BLK, c), lambda i: (i, 0))


def _rep(shape):
    return pl.BlockSpec(shape, lambda i: tuple(0 for _ in shape))


_GRID = _M // _BLK
_ARB = pltpu.CompilerParams(dimension_semantics=("arbitrary",))


def kernel(num_point, f0, f1, f2, f3, f4, FPS_0, FPS_1, FPS_2, FPS_3,
           W04, b04, W14, b14, W24, b24, W34, b34, W4, b4):
    g0, g1, g2, g3 = _sc_gather()(
        FPS_0, FPS_1, FPS_2, FPS_3,
        f0.reshape(_B * _N0, _C0), f1.reshape(_B * _N1, _C1),
        f2.reshape(_B * _N2, _C2), f3.reshape(_B * _N3, _C3))

    b04r, b14r = b04.reshape(1, _C4), b14.reshape(1, _C4)
    b24r, b34r = b24.reshape(1, _C4), b34.reshape(1, _C4)
    b4r = b4.reshape(1, _C4)
    f4f = f4.reshape(_M, _C4)

    st = pl.pallas_call(
        _stats_body,
        grid=(_GRID,),
        in_specs=[_rows(_C0), _rows(_C1), _rows(_C2), _rows(_C3),
                  _rep((_C4, _C0)), _rep((_C4, _C1)),
                  _rep((_C4, _C2)), _rep((_C4, _C3)),
                  _rep((1, _C4)), _rep((1, _C4)),
                  _rep((1, _C4)), _rep((1, _C4))],
        out_specs=_rep((8, _C4)),
        out_shape=jax.ShapeDtypeStruct((8, _C4), jnp.float32),
        compiler_params=_ARB,
    )(g0, g1, g2, g3, W04, W14, W24, W34, b04r, b14r, b24r, b34r)

    z5, st5 = pl.pallas_call(
        _mid_body,
        grid=(_GRID,),
        in_specs=[_rep((8, _C4)),
                  _rows(_C0), _rows(_C1), _rows(_C2), _rows(_C3),
                  _rows(_C4),
                  _rep((_C4, _C0)), _rep((_C4, _C1)),
                  _rep((_C4, _C2)), _rep((_C4, _C3)),
                  _rep((1, _C4)), _rep((1, _C4)),
                  _rep((1, _C4)), _rep((1, _C4)),
                  _rep((_C4, _C4)), _rep((1, _C4))],
        out_specs=(_rows(_C4), _rep((2, _C4))),
        out_shape=(jax.ShapeDtypeStruct((_M, _C4), jnp.float32),
                   jax.ShapeDtypeStruct((2, _C4), jnp.float32)),
        compiler_params=_ARB,
    )(st, g0, g1, g2, g3, f4f, W04, W14, W24, W34,
      b04r, b14r, b24r, b34r, W4, b4r)

    out = pl.pallas_call(
        _fin_body,
        grid=(_GRID,),
        in_specs=[_rep((2, _C4)), _rows(_C4), _rows(_C4)],
        out_specs=_rows(_C4),
        out_shape=jax.ShapeDtypeStruct((_M, _C4), jnp.float32),
        compiler_params=_ARB,
    )(st5, z5, f4f)

    return out.reshape(_B, _S, _C4)


# speedup vs baseline: 5.9075x; 1.4650x over previous
import functools

import jax
import jax.numpy as jnp
from jax import lax
from jax.experimental import pallas as pl
from jax.experimental.pallas import tpu as pltpu
from jax.experimental.pallas import tpu_sc as plsc

_B = 128
_S = 128
_N0, _N1, _N2, _N3 = 2048, 1024, 512, 256
_C0, _C1, _C2, _C3, _C4 = 64, 128, 128, 256, 256
_M = _B * _S
_EPS = 1e-5
_NC, _NS = 2, 16
_NW = _NC * _NS
_BPW = _B // _NW
_ABLK = 8
_BLK = 2048


def _sc_gather():
    mesh = plsc.VectorSubcoreMesh(
        core_axis_name="c", subcore_axis_name="s",
        num_cores=_NC, num_subcores=_NS)

    @functools.partial(
        pl.kernel,
        mesh=mesh,
        compiler_params=pltpu.CompilerParams(needs_layout_passes=False),
        out_type=(
            jax.ShapeDtypeStruct((_B, _S), jnp.int32),
            jax.ShapeDtypeStruct((_M, _C1), jnp.float32),
            jax.ShapeDtypeStruct((_M, _C2), jnp.float32),
            jax.ShapeDtypeStruct((_M, _C3), jnp.float32),
        ),
        scratch_types=[
            pltpu.VMEM((_N1,), jnp.int32),
            pltpu.VMEM((_N2,), jnp.int32),
            pltpu.VMEM((_N3,), jnp.int32),
            pltpu.VMEM((_S,), jnp.int32),
            pltpu.VMEM((_S,), jnp.int32),
            pltpu.VMEM((_S,), jnp.int32),
            pltpu.VMEM((_S,), jnp.int32),
            pltpu.VMEM((_S,), jnp.int32),
            pltpu.VMEM((_S, _C1), jnp.float32),
            pltpu.VMEM((_S, _C2), jnp.float32),
            pltpu.VMEM((_S, _C3), jnp.float32),
            pltpu.SemaphoreType.DMA,
        ],
    )
    def body(fps0, fps1, fps2, fps3, f1, f2, f3,
             i04, g1, g2, g3,
             t0, t1, t2, t3, li0, gi1, gi2, gi3, r1, r2, r3, sem):
        wid = lax.axis_index("s") * _NC + lax.axis_index("c")

        def per_batch(j, carry):
            b = wid * _BPW + j
            pltpu.sync_copy(fps0.at[b], t0)
            pltpu.sync_copy(fps1.at[b], t1)
            pltpu.sync_copy(fps2.at[b], t2)
            pltpu.sync_copy(fps3.at[b], t3)

            def chain(i, c):
                s = pl.ds(i * 16, 16)
                v3 = t3[s]
                v2 = plsc.load_gather(t2, [v3])
                v1 = plsc.load_gather(t1, [v2])
                v0 = plsc.load_gather(t0, [v1])
                li0[s] = v0
                gi3[s] = v3 + b * _N3
                gi2[s] = v2 + b * _N2
                gi1[s] = v1 + b * _N1
                return c

            lax.fori_loop(0, _S // 16, chain, jnp.int32(0))

            c1 = pltpu.async_copy(f1.at[gi1], r1, sem)
            c2 = pltpu.async_copy(f2.at[gi2], r2, sem)
            c3 = pltpu.async_copy(f3.at[gi3], r3, sem)
            pltpu.sync_copy(li0, i04.at[b])
            c1.wait()
            c2.wait()
            c3.wait()
            row = b * _S
            pltpu.sync_copy(r1, g1.at[pl.ds(row, _S)])
            pltpu.sync_copy(r2, g2.at[pl.ds(row, _S)])
            pltpu.sync_copy(r3, g3.at[pl.ds(row, _S)])
            return carry

        lax.fori_loop(0, _BPW, per_batch, jnp.int32(0))

    return body


def _mm(x, w):
    return lax.dot_general(x, w, (((1,), (1,)), ((), ())),
                           preferred_element_type=jnp.float32)


def _stats_body(i04b, f0b, g1, g2, g3, w04, w14, w24, w34,
                b04, b14, b24, b34, g0o, st):
    i = pl.program_id(0)

    @pl.when(i == 0)
    def _init():
        st[...] = jnp.zeros_like(st)

    idx = i04b[...]
    f0v = f0b[...]
    parts = []
    for j in range(_ABLK):
        row = idx[j:j + 1, :]
        n_iota = lax.broadcasted_iota(jnp.int32, (_N0, _S), 0)
        oh = (n_iota == row).astype(jnp.float32)
        f0j = f0v[j * _N0:(j + 1) * _N0, :]
        parts.append(lax.dot_general(
            oh, f0j, (((0,), (0,)), ((), ())),
            preferred_element_type=jnp.float32))
    g0 = jnp.concatenate(parts, axis=0)
    g0o[...] = g0

    for k, (g, w, bb) in enumerate(((g0, w04, b04), (g1[...], w14, b14),
                                    (g2[...], w24, b24), (g3[...], w34, b34))):
        z = _mm(g, w[...]) + bb[...]
        st[k:k + 1, :] += jnp.sum(z, axis=0, keepdims=True)
        st[k + 4:k + 5, :] += jnp.sum(z * z, axis=0, keepdims=True)


def _mid_body(st, g0, g1, g2, g3, f4b, w04, w14, w24, w34,
              b04, b14, b24, b34, w4, b4, z5, st5):
    i = pl.program_id(0)

    @pl.when(i == 0)
    def _init():
        st5[...] = jnp.zeros_like(st5)

    stv = st[...]
    acc = f4b[...]
    for k, (g, w, bb) in enumerate(((g0, w04, b04), (g1, w14, b14),
                                    (g2, w24, b24), (g3, w34, b34))):
        z = _mm(g[...], w[...]) + bb[...]
        mu = stv[k:k + 1, :] * (1.0 / _M)
        ex2 = stv[k + 4:k + 5, :] * (1.0 / _M)
        rs = lax.rsqrt(ex2 - mu * mu + _EPS)
        y = (z - mu) * rs
        acc = acc + jnp.where(y > 0, y, 0.2 * y)
    z = _mm(acc, w4[...]) + b4[...]
    z5[...] = z
    st5[0:1, :] += jnp.sum(z, axis=0, keepdims=True)
    st5[1:2, :] += jnp.sum(z * z, axis=0, keepdims=True)


def _fin_body(st5, z5b, f4b, ob):
    stv = st5[...]
    mu = stv[0:1, :] * (1.0 / _M)
    ex2 = stv[1:2, :] * (1.0 / _M)
    rs = lax.rsqrt(ex2 - mu * mu + _EPS)
    y = (z5b[...] - mu) * rs
    ob[...] = jnp.where(y > 0, y, 0.2 * y) + f4b[...]


def _rows(c, blk=_BLK):
    return pl.BlockSpec((blk, c), lambda i: (i, 0))


def _rep(shape):
    return pl.BlockSpec(shape, lambda i: tuple(0 for _ in shape))


_ARB = pltpu.CompilerParams(dimension_semantics=("arbitrary",))


def kernel(num_point, f0, f1, f2, f3, f4, FPS_0, FPS_1, FPS_2, FPS_3,
           W04, b04, W14, b14, W24, b24, W34, b34, W4, b4):
    i04, g1, g2, g3 = _sc_gather()(
        FPS_0, FPS_1, FPS_2, FPS_3,
        f1.reshape(_B * _N1, _C1),
        f2.reshape(_B * _N2, _C2), f3.reshape(_B * _N3, _C3))

    b04r, b14r = b04.reshape(1, _C4), b14.reshape(1, _C4)
    b24r, b34r = b24.reshape(1, _C4), b34.reshape(1, _C4)
    b4r = b4.reshape(1, _C4)
    f4f = f4.reshape(_M, _C4)
    f0f = f0.reshape(_B * _N0, _C0)
    arows = _ABLK * _S

    g0, st = pl.pallas_call(
        _stats_body,
        grid=(_B // _ABLK,),
        in_specs=[_rows(_S, _ABLK), _rows(_C0, _ABLK * _N0),
                  _rows(_C1, arows), _rows(_C2, arows), _rows(_C3, arows),
                  _rep((_C4, _C0)), _rep((_C4, _C1)),
                  _rep((_C4, _C2)), _rep((_C4, _C3)),
                  _rep((1, _C4)), _rep((1, _C4)),
                  _rep((1, _C4)), _rep((1, _C4))],
        out_specs=(_rows(_C0, arows), _rep((8, _C4))),
        out_shape=(jax.ShapeDtypeStruct((_M, _C0), jnp.float32),
                   jax.ShapeDtypeStruct((8, _C4), jnp.float32)),
        compiler_params=_ARB,
    )(i04, f0f, g1, g2, g3, W04, W14, W24, W34, b04r, b14r, b24r, b34r)

    z5, st5 = pl.pallas_call(
        _mid_body,
        grid=(_M // _BLK,),
        in_specs=[_rep((8, _C4)),
                  _rows(_C0), _rows(_C1), _rows(_C2), _rows(_C3),
                  _rows(_C4),
                  _rep((_C4, _C0)), _rep((_C4, _C1)),
                  _rep((_C4, _C2)), _rep((_C4, _C3)),
                  _rep((1, _C4)), _rep((1, _C4)),
                  _rep((1, _C4)), _rep((1, _C4)),
                  _rep((_C4, _C4)), _rep((1, _C4))],
        out_specs=(_rows(_C4), _rep((2, _C4))),
        out_shape=(jax.ShapeDtypeStruct((_M, _C4), jnp.float32),
                   jax.ShapeDtypeStruct((2, _C4), jnp.float32)),
        compiler_params=_ARB,
    )(st, g0, g1, g2, g3, f4f, W04, W14, W24, W34,
      b04r, b14r, b24r, b34r, W4, b4r)

    out = pl.pallas_call(
        _fin_body,
        grid=(_M // _BLK,),
        in_specs=[_rep((2, _C4)), _rows(_C4), _rows(_C4)],
        out_specs=_rows(_C4),
        out_shape=jax.ShapeDtypeStruct((_M, _C4), jnp.float32),
        compiler_params=_ARB,
    )(st5, z5, f4f)

    return out.reshape(_B, _S, _C4)


# speedup vs baseline: 8.4356x; 1.4279x over previous
import functools

import jax
import jax.numpy as jnp
from jax import lax
from jax.experimental import pallas as pl
from jax.experimental.pallas import tpu as pltpu
from jax.experimental.pallas import tpu_sc as plsc

_B = 128
_S = 128
_N0, _N1, _N2, _N3 = 2048, 1024, 512, 256
_C0, _C1, _C2, _C3, _C4 = 64, 128, 128, 256, 256
_M = _B * _S
_EPS = 1e-5
_NC, _NS = 2, 16
_NW = _NC * _NS
_BPW = _B // _NW
_ABLK = 8
_BLK = 2048

_SC_PARAMS = pltpu.CompilerParams(needs_layout_passes=False)


def _sc_mesh():
    return plsc.VectorSubcoreMesh(
        core_axis_name="c", subcore_axis_name="s",
        num_cores=_NC, num_subcores=_NS)


def _sc_chain():
    @functools.partial(
        pl.kernel,
        mesh=_sc_mesh(),
        compiler_params=_SC_PARAMS,
        out_type=(
            jax.ShapeDtypeStruct((_B, _S), jnp.int32),
            jax.ShapeDtypeStruct((_B, _S), jnp.int32),
            jax.ShapeDtypeStruct((_B, _S), jnp.int32),
            jax.ShapeDtypeStruct((_B, _S), jnp.int32),
        ),
        scratch_types=[
            pltpu.VMEM((_N1,), jnp.int32),
            pltpu.VMEM((_N2,), jnp.int32),
            pltpu.VMEM((_N3,), jnp.int32),
            pltpu.VMEM((_S,), jnp.int32),
            pltpu.VMEM((_S,), jnp.int32),
            pltpu.VMEM((_S,), jnp.int32),
            pltpu.VMEM((_S,), jnp.int32),
            pltpu.VMEM((_S,), jnp.int32),
        ],
    )
    def body(fps0, fps1, fps2, fps3,
             i04, o1, o2, o3,
             t0, t1, t2, t3, li0, gi1, gi2, gi3):
        wid = lax.axis_index("s") * _NC + lax.axis_index("c")

        def per_batch(j, carry):
            b = wid * _BPW + j
            pltpu.sync_copy(fps0.at[b], t0)
            pltpu.sync_copy(fps1.at[b], t1)
            pltpu.sync_copy(fps2.at[b], t2)
            pltpu.sync_copy(fps3.at[b], t3)

            def chain(i, c):
                s = pl.ds(i * 16, 16)
                v3 = t3[s]
                v2 = plsc.load_gather(t2, [v3])
                v1 = plsc.load_gather(t1, [v2])
                v0 = plsc.load_gather(t0, [v1])
                li0[s] = v0
                gi3[s] = v3 + b * _N3
                gi2[s] = v2 + b * _N2
                gi1[s] = v1 + b * _N1
                return c

            lax.fori_loop(0, _S // 16, chain, jnp.int32(0))
            pltpu.sync_copy(li0, i04.at[b])
            pltpu.sync_copy(gi1, o1.at[b])
            pltpu.sync_copy(gi2, o2.at[b])
            pltpu.sync_copy(gi3, o3.at[b])
            return carry

        lax.fori_loop(0, _BPW, per_batch, jnp.int32(0))

    return body


def _sc_gather():
    @functools.partial(
        pl.kernel,
        mesh=_sc_mesh(),
        compiler_params=_SC_PARAMS,
        out_type=(
            jax.ShapeDtypeStruct((_M, _C1), jnp.float32),
            jax.ShapeDtypeStruct((_M, _C2), jnp.float32),
            jax.ShapeDtypeStruct((_M, _C3), jnp.float32),
        ),
        scratch_types=[
            pltpu.VMEM((_S,), jnp.int32),
            pltpu.VMEM((_S,), jnp.int32),
            pltpu.VMEM((_S,), jnp.int32),
            pltpu.VMEM((_S, _C1), jnp.float32),
            pltpu.VMEM((_S, _C2), jnp.float32),
            pltpu.VMEM((_S, _C3), jnp.float32),
            pltpu.SemaphoreType.DMA,
        ],
    )
    def body(o1, o2, o3, f1, f2, f3,
             g1, g2, g3,
             gi1, gi2, gi3, r1, r2, r3, sem):
        wid = lax.axis_index("s") * _NC + lax.axis_index("c")

        def per_batch(j, carry):
            b = wid * _BPW + j
            pltpu.sync_copy(o1.at[b], gi1)
            pltpu.sync_copy(o2.at[b], gi2)
            pltpu.sync_copy(o3.at[b], gi3)
            c1 = pltpu.async_copy(f1.at[gi1], r1, sem)
            c2 = pltpu.async_copy(f2.at[gi2], r2, sem)
            c3 = pltpu.async_copy(f3.at[gi3], r3, sem)
            c1.wait()
            c2.wait()
            c3.wait()
            row = b * _S
            pltpu.sync_copy(r1, g1.at[pl.ds(row, _S)])
            pltpu.sync_copy(r2, g2.at[pl.ds(row, _S)])
            pltpu.sync_copy(r3, g3.at[pl.ds(row, _S)])
            return carry

        lax.fori_loop(0, _BPW, per_batch, jnp.int32(0))

    return body


def _mm(x, w):
    return lax.dot_general(x, w, (((1,), (1,)), ((), ())),
                           preferred_element_type=jnp.float32)


def _a0_body(i04b, f0tb, w04, b04, g0o, st0):
    i = pl.program_id(0)

    @pl.when(i == 0)
    def _init():
        st0[...] = jnp.zeros_like(st0)

    idx = i04b[...]
    f0t = f0tb[...]
    parts = []
    for j in range(_ABLK):
        row = idx[j:j + 1, :]
        n_iota = lax.broadcasted_iota(jnp.int32, (_N0, _S), 0)
        oh = (n_iota == row).astype(jnp.float32)
        parts.append(lax.dot_general(
            oh, f0t[j], (((0,), (1,)), ((), ())),
            preferred_element_type=jnp.float32))
    g0 = jnp.concatenate(parts, axis=0)
    g0o[...] = g0
    z = _mm(g0, w04[...]) + b04[...]
    st0[0:1, :] += jnp.sum(z, axis=0, keepdims=True)
    st0[1:2, :] += jnp.sum(z * z, axis=0, keepdims=True)


def _a1_body(g1, g2, g3, w14, w24, w34, b14, b24, b34, st):
    i = pl.program_id(0)

    @pl.when(i == 0)
    def _init():
        st[...] = jnp.zeros_like(st)

    for k, (g, w, bb) in enumerate(((g1, w14, b14),
                                    (g2, w24, b24), (g3, w34, b34))):
        z = _mm(g[...], w[...]) + bb[...]
        st[k:k + 1, :] += jnp.sum(z, axis=0, keepdims=True)
        st[k + 3:k + 4, :] += jnp.sum(z * z, axis=0, keepdims=True)


def _mid_body(st0, st, g0, g1, g2, g3, f4b, w04, w14, w24, w34,
              b04, b14, b24, b34, w4, b4, z5, st5):
    i = pl.program_id(0)

    @pl.when(i == 0)
    def _init():
        st5[...] = jnp.zeros_like(st5)

    st0v = st0[...]
    stv = st[...]
    acc = f4b[...]
    branches = ((g0, w04, b04, st0v[0:1, :], st0v[1:2, :]),
                (g1, w14, b14, stv[0:1, :], stv[3:4, :]),
                (g2, w24, b24, stv[1:2, :], stv[4:5, :]),
                (g3, w34, b34, stv[2:3, :], stv[5:6, :]))
    for g, w, bb, sm, sq in branches:
        z = _mm(g[...], w[...]) + bb[...]
        mu = sm * (1.0 / _M)
        ex2 = sq * (1.0 / _M)
        rs = lax.rsqrt(ex2 - mu * mu + _EPS)
        y = (z - mu) * rs
        acc = acc + jnp.where(y > 0, y, 0.2 * y)
    z = _mm(acc, w4[...]) + b4[...]
    z5[...] = z
    st5[0:1, :] += jnp.sum(z, axis=0, keepdims=True)
    st5[1:2, :] += jnp.sum(z * z, axis=0, keepdims=True)


def _fin_body(st5, z5b, f4b, ob):
    stv = st5[...]
    mu = stv[0:1, :] * (1.0 / _M)
    ex2 = stv[1:2, :] * (1.0 / _M)
    rs = lax.rsqrt(ex2 - mu * mu + _EPS)
    y = (z5b[...] - mu) * rs
    ob[...] = jnp.where(y > 0, y, 0.2 * y) + f4b[...]


def _rows(c, blk=_BLK):
    return pl.BlockSpec((blk, c), lambda i: (i, 0))


def _rep(shape):
    return pl.BlockSpec(shape, lambda i: tuple(0 for _ in shape))


_ARB = pltpu.CompilerParams(dimension_semantics=("arbitrary",))


def kernel(num_point, f0, f1, f2, f3, f4, FPS_0, FPS_1, FPS_2, FPS_3,
           W04, b04, W14, b14, W24, b24, W34, b34, W4, b4):
    i04, o1, o2, o3 = _sc_chain()(FPS_0, FPS_1, FPS_2, FPS_3)
    g1, g2, g3 = _sc_gather()(
        o1, o2, o3,
        f1.reshape(_B * _N1, _C1),
        f2.reshape(_B * _N2, _C2), f3.reshape(_B * _N3, _C3))

    b04r, b14r = b04.reshape(1, _C4), b14.reshape(1, _C4)
    b24r, b34r = b24.reshape(1, _C4), b34.reshape(1, _C4)
    b4r = b4.reshape(1, _C4)
    f4f = f4.reshape(_M, _C4)
    f0t = jnp.transpose(f0, (0, 2, 1))
    arows = _ABLK * _S

    g0, st0 = pl.pallas_call(
        _a0_body,
        grid=(_B // _ABLK,),
        in_specs=[_rows(_S, _ABLK),
                  pl.BlockSpec((_ABLK, _C0, _N0), lambda i: (i, 0, 0)),
                  _rep((_C4, _C0)), _rep((1, _C4))],
        out_specs=(_rows(_C0, arows), _rep((2, _C4))),
        out_shape=(jax.ShapeDtypeStruct((_M, _C0), jnp.float32),
                   jax.ShapeDtypeStruct((2, _C4), jnp.float32)),
        compiler_params=_ARB,
    )(i04, f0t, W04, b04r)

    st = pl.pallas_call(
        _a1_body,
        grid=(_M // _BLK,),
        in_specs=[_rows(_C1), _rows(_C2), _rows(_C3),
                  _rep((_C4, _C1)), _rep((_C4, _C2)), _rep((_C4, _C3)),
                  _rep((1, _C4)), _rep((1, _C4)), _rep((1, _C4))],
        out_specs=_rep((6, _C4)),
        out_shape=jax.ShapeDtypeStruct((6, _C4), jnp.float32),
        compiler_params=_ARB,
    )(g1, g2, g3, W14, W24, W34, b14r, b24r, b34r)

    z5, st5 = pl.pallas_call(
        _mid_body,
        grid=(_M // _BLK,),
        in_specs=[_rep((2, _C4)), _rep((6, _C4)),
                  _rows(_C0), _rows(_C1), _rows(_C2), _rows(_C3),
                  _rows(_C4),
                  _rep((_C4, _C0)), _rep((_C4, _C1)),
                  _rep((_C4, _C2)), _rep((_C4, _C3)),
                  _rep((1, _C4)), _rep((1, _C4)),
                  _rep((1, _C4)), _rep((1, _C4)),
                  _rep((_C4, _C4)), _rep((1, _C4))],
        out_specs=(_rows(_C4), _rep((2, _C4))),
        out_shape=(jax.ShapeDtypeStruct((_M, _C4), jnp.float32),
                   jax.ShapeDtypeStruct((2, _C4), jnp.float32)),
        compiler_params=_ARB,
    )(st0, st, g0, g1, g2, g3, f4f, W04, W14, W24, W34,
      b04r, b14r, b24r, b34r, W4, b4r)

    out = pl.pallas_call(
        _fin_body,
        grid=(_M // _BLK,),
        in_specs=[_rep((2, _C4)), _rows(_C4), _rows(_C4)],
        out_specs=_rows(_C4),
        out_shape=jax.ShapeDtypeStruct((_M, _C4), jnp.float32),
        compiler_params=_ARB,
    )(st5, z5, f4f)

    return out.reshape(_B, _S, _C4)


# speedup vs baseline: 9.1076x; 1.0797x over previous
import functools

import jax
import jax.numpy as jnp
from jax import lax
from jax.experimental import pallas as pl
from jax.experimental.pallas import tpu as pltpu
from jax.experimental.pallas import tpu_sc as plsc

_B = 128
_S = 128
_N0, _N1, _N2, _N3 = 2048, 1024, 512, 256
_C0, _C1, _C2, _C3, _C4 = 64, 128, 128, 256, 256
_M = _B * _S
_EPS = 1e-5
_NC, _NS = 2, 16
_NW = _NC * _NS
_BPW = _B // _NW
_ABLK = 8
_BLK = 2048

_SC_PARAMS = pltpu.CompilerParams(needs_layout_passes=False)


def _sc_mesh():
    return plsc.VectorSubcoreMesh(
        core_axis_name="c", subcore_axis_name="s",
        num_cores=_NC, num_subcores=_NS)


def _sc_chain():
    @functools.partial(
        pl.kernel,
        mesh=_sc_mesh(),
        compiler_params=_SC_PARAMS,
        out_type=(
            jax.ShapeDtypeStruct((_B, _S), jnp.int32),
            jax.ShapeDtypeStruct((_B, _S), jnp.int32),
            jax.ShapeDtypeStruct((_B, _S), jnp.int32),
            jax.ShapeDtypeStruct((_B, _S), jnp.int32),
        ),
        scratch_types=[
            pltpu.VMEM((_N1,), jnp.int32),
            pltpu.VMEM((_N2,), jnp.int32),
            pltpu.VMEM((_N3,), jnp.int32),
            pltpu.VMEM((_S,), jnp.int32),
            pltpu.VMEM((_S,), jnp.int32),
            pltpu.VMEM((_S,), jnp.int32),
            pltpu.VMEM((_S,), jnp.int32),
            pltpu.VMEM((_S,), jnp.int32),
        ],
    )
    def body(fps0, fps1, fps2, fps3,
             i04, o1, o2, o3,
             t0, t1, t2, t3, li0, gi1, gi2, gi3):
        wid = lax.axis_index("s") * _NC + lax.axis_index("c")

        def per_batch(j, carry):
            b = wid * _BPW + j
            pltpu.sync_copy(fps0.at[b], t0)
            pltpu.sync_copy(fps1.at[b], t1)
            pltpu.sync_copy(fps2.at[b], t2)
            pltpu.sync_copy(fps3.at[b], t3)

            def chain(i, c):
                s = pl.ds(i * 16, 16)
                v3 = t3[s]
                v2 = plsc.load_gather(t2, [v3])
                v1 = plsc.load_gather(t1, [v2])
                v0 = plsc.load_gather(t0, [v1])
                li0[s] = v0
                gi3[s] = v3 + b * _N3
                gi2[s] = v2 + b * _N2
                gi1[s] = v1 + b * _N1
                return c

            lax.fori_loop(0, _S // 16, chain, jnp.int32(0))
            pltpu.sync_copy(li0, i04.at[b])
            pltpu.sync_copy(gi1, o1.at[b])
            pltpu.sync_copy(gi2, o2.at[b])
            pltpu.sync_copy(gi3, o3.at[b])
            return carry

        lax.fori_loop(0, _BPW, per_batch, jnp.int32(0))

    return body


def _sc_gather():
    @functools.partial(
        pl.kernel,
        mesh=_sc_mesh(),
        compiler_params=_SC_PARAMS,
        out_type=(
            jax.ShapeDtypeStruct((_M, _C1), jnp.float32),
            jax.ShapeDtypeStruct((_M, _C2), jnp.float32),
            jax.ShapeDtypeStruct((_M, _C3), jnp.float32),
        ),
        scratch_types=[
            pltpu.VMEM((_S,), jnp.int32),
            pltpu.VMEM((_S,), jnp.int32),
            pltpu.VMEM((_S,), jnp.int32),
            pltpu.VMEM((_S, _C1), jnp.float32),
            pltpu.VMEM((_S, _C2), jnp.float32),
            pltpu.VMEM((_S, _C3), jnp.float32),
            pltpu.SemaphoreType.DMA,
        ],
    )
    def body(o1, o2, o3, f1, f2, f3,
             g1, g2, g3,
             gi1, gi2, gi3, r1, r2, r3, sem):
        wid = lax.axis_index("s") * _NC + lax.axis_index("c")

        def per_batch(j, carry):
            b = wid * _BPW + j
            pltpu.sync_copy(o1.at[b], gi1)
            pltpu.sync_copy(o2.at[b], gi2)
            pltpu.sync_copy(o3.at[b], gi3)
            c1 = pltpu.async_copy(f1.at[gi1], r1, sem)
            c2 = pltpu.async_copy(f2.at[gi2], r2, sem)
            c3 = pltpu.async_copy(f3.at[gi3], r3, sem)
            c1.wait()
            c2.wait()
            c3.wait()
            row = b * _S
            pltpu.sync_copy(r1, g1.at[pl.ds(row, _S)])
            pltpu.sync_copy(r2, g2.at[pl.ds(row, _S)])
            pltpu.sync_copy(r3, g3.at[pl.ds(row, _S)])
            return carry

        lax.fori_loop(0, _BPW, per_batch, jnp.int32(0))

    return body


def _mm(x, w):
    return lax.dot_general(x, w, (((1,), (1,)), ((), ())),
                           preferred_element_type=jnp.float32)


def _a0_body(i04b, f0tb, w04, b04, g0o, st0):
    i = pl.program_id(0)

    @pl.when(i == 0)
    def _init():
        st0[...] = jnp.zeros_like(st0)

    idx = i04b[...]
    f0t = f0tb[...]
    parts = []
    for j in range(_ABLK):
        row = idx[j:j + 1, :]
        n_iota = lax.broadcasted_iota(jnp.int32, (_N0, _S), 0)
        oh = (n_iota == row).astype(jnp.float32)
        parts.append(lax.dot_general(
            oh, f0t[j], (((0,), (1,)), ((), ())),
            preferred_element_type=jnp.float32))
    g0 = jnp.concatenate(parts, axis=0)
    g0o[...] = g0
    z = _mm(g0, w04[...]) + b04[...]
    st0[0:1, :] += jnp.sum(z, axis=0, keepdims=True)
    st0[1:2, :] += jnp.sum(z * z, axis=0, keepdims=True)


def _norm_leaky(z, sm, sq):
    mu = sm * (1.0 / _M)
    ex2 = sq * (1.0 / _M)
    rs = lax.rsqrt(ex2 - mu * mu + _EPS)
    y = (z - mu) * rs
    return jnp.where(y > 0, y, 0.2 * y)


def _dense_body(st0, g0b, g1b, g2b, g3b, f4b,
                w04, w14, w24, w34, b04, b14, b24, b34, w4, b4,
                ob,
                sg1, sg2, sz5, st, st5, stage, sem):
    p = pl.program_id(0)
    i = pl.program_id(1)
    r = pl.ds(i * _BLK, _BLK)

    @pl.when((p == 0) & (i == 0))
    def _init():
        st[...] = jnp.zeros_like(st)
        st5[...] = jnp.zeros_like(st5)

    @pl.when(p == 0)
    def _phase0():
        for k, (gv, w, bb) in enumerate(
                ((g1b[...], w14, b14), (g2b[...], w24, b24),
                 (g3b[...], w34, b34))):
            if k == 0:
                sg1[r, :] = gv
            elif k == 1:
                sg2[r, :] = gv
            z = _mm(gv, w[...]) + bb[...]
            st[k:k + 1, :] += jnp.sum(z, axis=0, keepdims=True)
            st[k + 3:k + 4, :] += jnp.sum(z * z, axis=0, keepdims=True)

    @pl.when(p == 1)
    def _phase1():
        st0v = st0[...]
        stv = st[...]
        acc = f4b[...]
        branches = ((g0b[...], w04, b04, st0v[0:1, :], st0v[1:2, :]),
                    (sg1[r, :], w14, b14, stv[0:1, :], stv[3:4, :]),
                    (sg2[r, :], w24, b24, stv[1:2, :], stv[4:5, :]),
                    (g3b[...], w34, b34, stv[2:3, :], stv[5:6, :]))
        for gv, w, bb, sm, sq in branches:
            z = _mm(gv, w[...]) + bb[...]
            acc = acc + _norm_leaky(z, sm, sq)
        z = _mm(acc, w4[...]) + b4[...]
        sz5[r, :] = z
        st5[0:1, :] += jnp.sum(z, axis=0, keepdims=True)
        st5[1:2, :] += jnp.sum(z * z, axis=0, keepdims=True)

    @pl.when(p == 2)
    def _phase2():
        stv = st5[...]
        stage[...] = (_norm_leaky(sz5[r, :], stv[0:1, :], stv[1:2, :])
                      + f4b[...])
        pltpu.async_copy(stage, ob.at[r, :], sem).wait()


def _rows(c, blk=_BLK):
    return pl.BlockSpec((blk, c), lambda i: (i, 0))


def _rep(shape):
    return pl.BlockSpec(shape, lambda i: tuple(0 for _ in shape))


_ARB = pltpu.CompilerParams(dimension_semantics=("arbitrary",))


def kernel(num_point, f0, f1, f2, f3, f4, FPS_0, FPS_1, FPS_2, FPS_3,
           W04, b04, W14, b14, W24, b24, W34, b34, W4, b4):
    i04, o1, o2, o3 = _sc_chain()(FPS_0, FPS_1, FPS_2, FPS_3)
    g1, g2, g3 = _sc_gather()(
        o1, o2, o3,
        f1.reshape(_B * _N1, _C1),
        f2.reshape(_B * _N2, _C2), f3.reshape(_B * _N3, _C3))

    b04r, b14r = b04.reshape(1, _C4), b14.reshape(1, _C4)
    b24r, b34r = b24.reshape(1, _C4), b34.reshape(1, _C4)
    b4r = b4.reshape(1, _C4)
    f4f = f4.reshape(_M, _C4)
    f0t = jnp.transpose(f0, (0, 2, 1))
    arows = _ABLK * _S

    g0, st0 = pl.pallas_call(
        _a0_body,
        grid=(_B // _ABLK,),
        in_specs=[_rows(_S, _ABLK),
                  pl.BlockSpec((_ABLK, _C0, _N0), lambda i: (i, 0, 0)),
                  _rep((_C4, _C0)), _rep((1, _C4))],
        out_specs=(_rows(_C0, arows), _rep((2, _C4))),
        out_shape=(jax.ShapeDtypeStruct((_M, _C0), jnp.float32),
                   jax.ShapeDtypeStruct((2, _C4), jnp.float32)),
        compiler_params=_ARB,
    )(i04, f0t, W04, b04r)

    def _prow(c, cond):
        return pl.BlockSpec(
            (_BLK, c), lambda p, i: (jnp.where(cond(p), i, 0), 0))

    def _prep(shape):
        return pl.BlockSpec(shape, lambda p, i: tuple(0 for _ in shape))

    out = pl.pallas_call(
        _dense_body,
        grid=(3, _M // _BLK),
        in_specs=[_prep((2, _C4)),
                  _prow(_C0, lambda p: p == 1),
                  _prow(_C1, lambda p: p == 0),
                  _prow(_C2, lambda p: p == 0),
                  _prow(_C3, lambda p: p != 2),
                  _prow(_C4, lambda p: p != 0),
                  _prep((_C4, _C0)), _prep((_C4, _C1)),
                  _prep((_C4, _C2)), _prep((_C4, _C3)),
                  _prep((1, _C4)), _prep((1, _C4)),
                  _prep((1, _C4)), _prep((1, _C4)),
                  _prep((_C4, _C4)), _prep((1, _C4))],
        out_specs=pl.BlockSpec(memory_space=pl.ANY),
        out_shape=jax.ShapeDtypeStruct((_M, _C4), jnp.float32),
        scratch_shapes=[
            pltpu.VMEM((_M, _C1), jnp.float32),
            pltpu.VMEM((_M, _C2), jnp.float32),
            pltpu.VMEM((_M, _C4), jnp.float32),
            pltpu.VMEM((6, _C4), jnp.float32),
            pltpu.VMEM((2, _C4), jnp.float32),
            pltpu.VMEM((_BLK, _C4), jnp.float32),
            pltpu.SemaphoreType.DMA,
        ],
        compiler_params=pltpu.CompilerParams(
            dimension_semantics=("arbitrary", "arbitrary"),
            vmem_limit_bytes=120 * 1024 * 1024),
    )(st0, g0, g1, g2, g3, f4f, W04, W14, W24, W34,
      b04r, b14r, b24r, b34r, W4, b4r)

    return out.reshape(_B, _S, _C4)


# speedup vs baseline: 9.3620x; 1.0279x over previous
import functools

import jax
import jax.numpy as jnp
from jax import lax
from jax.experimental import pallas as pl
from jax.experimental.pallas import tpu as pltpu
from jax.experimental.pallas import tpu_sc as plsc

_B = 128
_S = 128
_N0, _N1, _N2, _N3 = 2048, 1024, 512, 256
_C0, _C1, _C2, _C3, _C4 = 64, 128, 128, 256, 256
_M = _B * _S
_EPS = 1e-5
_NC, _NS = 2, 16
_NW = _NC * _NS
_BPW = _B // _NW
_ABLK = 8
_BLK = 2048
_CH = 64

_SC_PARAMS = pltpu.CompilerParams(needs_layout_passes=False)


def _sc_mesh():
    return plsc.VectorSubcoreMesh(
        core_axis_name="c", subcore_axis_name="s",
        num_cores=_NC, num_subcores=_NS)


def _sc_chain():
    @functools.partial(
        pl.kernel,
        mesh=_sc_mesh(),
        compiler_params=_SC_PARAMS,
        out_type=(
            jax.ShapeDtypeStruct((_B, _S), jnp.int32),
            jax.ShapeDtypeStruct((_B, _S), jnp.int32),
            jax.ShapeDtypeStruct((_B, _S), jnp.int32),
            jax.ShapeDtypeStruct((_B, _S), jnp.int32),
        ),
        scratch_types=[
            pltpu.VMEM((_BPW, _N1), jnp.int32),
            pltpu.VMEM((_BPW, _N2), jnp.int32),
            pltpu.VMEM((_BPW, _N3), jnp.int32),
            pltpu.VMEM((_BPW, _S), jnp.int32),
            pltpu.VMEM((_BPW, _S), jnp.int32),
            pltpu.VMEM((_BPW, _S), jnp.int32),
            pltpu.VMEM((_BPW, _S), jnp.int32),
            pltpu.VMEM((_BPW, _S), jnp.int32),
        ],
    )
    def body(fps0, fps1, fps2, fps3,
             i04, o1, o2, o3,
             t0, t1, t2, t3, li0, gi1, gi2, gi3):
        wid = lax.axis_index("s") * _NC + lax.axis_index("c")
        b0 = wid * _BPW
        rows = pl.ds(b0, _BPW)
        pltpu.sync_copy(fps0.at[rows], t0)
        pltpu.sync_copy(fps1.at[rows], t1)
        pltpu.sync_copy(fps2.at[rows], t2)
        pltpu.sync_copy(fps3.at[rows], t3)

        for j in range(_BPW):
            jv = jnp.full((16,), j, jnp.int32)

            def chain(i, c, j=j, jv=jv):
                s = pl.ds(i * 16, 16)
                v3 = t3[j, s]
                v2 = plsc.load_gather(t2, [jv, v3])
                v1 = plsc.load_gather(t1, [jv, v2])
                v0 = plsc.load_gather(t0, [jv, v1])
                li0[j, s] = v0
                gi3[j, s] = v3 + (b0 + j) * _N3
                gi2[j, s] = v2 + (b0 + j) * _N2
                gi1[j, s] = v1 + (b0 + j) * _N1
                return c

            lax.fori_loop(0, _S // 16, chain, jnp.int32(0))

        pltpu.sync_copy(li0, i04.at[rows])
        pltpu.sync_copy(gi1, o1.at[rows])
        pltpu.sync_copy(gi2, o2.at[rows])
        pltpu.sync_copy(gi3, o3.at[rows])

    return body


def _sc_gather():
    @functools.partial(
        pl.kernel,
        mesh=_sc_mesh(),
        compiler_params=_SC_PARAMS,
        out_type=(
            jax.ShapeDtypeStruct((_M, _C1), jnp.float32),
            jax.ShapeDtypeStruct((_M, _C2), jnp.float32),
            jax.ShapeDtypeStruct((_M, _C3), jnp.float32),
        ),
        scratch_types=[
            pltpu.VMEM((_BPW, _S), jnp.int32),
            pltpu.VMEM((_BPW, _S), jnp.int32),
            pltpu.VMEM((_BPW, _S), jnp.int32),
            pltpu.VMEM((_CH, _C1), jnp.float32),
            pltpu.VMEM((_CH, _C2), jnp.float32),
            pltpu.VMEM((_CH, _C3), jnp.float32),
            pltpu.VMEM((_CH, _C1), jnp.float32),
            pltpu.VMEM((_CH, _C2), jnp.float32),
            pltpu.VMEM((_CH, _C3), jnp.float32),
            pltpu.SemaphoreType.DMA,
            pltpu.SemaphoreType.DMA,
            pltpu.SemaphoreType.DMA,
            pltpu.SemaphoreType.DMA,
        ],
    )
    def body(o1, o2, o3, f1, f2, f3,
             g1, g2, g3,
             i1b, i2b, i3b, r1a, r2a, r3a, r1b, r2b, r3b,
             gsa, gsb, wsa, wsb):
        wid = lax.axis_index("s") * _NC + lax.axis_index("c")
        b0 = wid * _BPW
        rows = pl.ds(b0, _BPW)
        pltpu.sync_copy(o1.at[rows], i1b)
        pltpu.sync_copy(o2.at[rows], i2b)
        pltpu.sync_copy(o3.at[rows], i3b)

        bufs = ((r1a, r2a, r3a), (r1b, r2b, r3b))
        gsems = (gsa, gsb)
        wsems = (wsa, wsb)
        nch = _BPW * _S // _CH

        def start_g(c):
            s = c % 2
            bj, h = divmod(c * _CH, _S)
            idx = (i1b.at[bj, pl.ds(h, _CH)], i2b.at[bj, pl.ds(h, _CH)],
                   i3b.at[bj, pl.ds(h, _CH)])
            return tuple(
                pltpu.async_copy(f.at[ix], buf, gsems[s])
                for f, ix, buf in zip((f1, f2, f3), idx, bufs[s]))

        def start_w(c):
            s = c % 2
            row = b0 * _S + c * _CH
            return tuple(
                pltpu.async_copy(buf, g.at[pl.ds(row, _CH)], wsems[s])
                for g, buf in zip((g1, g2, g3), bufs[s]))

        pg = {0: start_g(0)}
        pw = {}
        for c in range(1, nch + 1):
            if c <= nch - 1:
                if c >= 2:
                    for w in pw.pop(c - 2):
                        w.wait()
                pg[c] = start_g(c)
            for g in pg.pop(c - 1):
                g.wait()
            pw[c - 1] = start_w(c - 1)
        for w in pw[nch - 2]:
            w.wait()
        for w in pw[nch - 1]:
            w.wait()

    return body


def _mm(x, w):
    return lax.dot_general(x, w, (((1,), (1,)), ((), ())),
                           preferred_element_type=jnp.float32)


def _a0_body(i04b, f0tb, w04, b04, g0o, st0):
    i = pl.program_id(0)

    @pl.when(i == 0)
    def _init():
        st0[...] = jnp.zeros_like(st0)

    idx = i04b[...]
    f0t = f0tb[...]
    parts = []
    for j in range(_ABLK):
        row = idx[j:j + 1, :]
        n_iota = lax.broadcasted_iota(jnp.int32, (_N0, _S), 0)
        oh = (n_iota == row).astype(jnp.float32)
        parts.append(lax.dot_general(
            oh, f0t[j], (((0,), (1,)), ((), ())),
            preferred_element_type=jnp.float32))
    g0 = jnp.concatenate(parts, axis=0)
    g0o[...] = g0
    z = _mm(g0, w04[...]) + b04[...]
    st0[0:1, :] += jnp.sum(z, axis=0, keepdims=True)
    st0[1:2, :] += jnp.sum(z * z, axis=0, keepdims=True)


def _norm_leaky(z, sm, sq):
    mu = sm * (1.0 / _M)
    ex2 = sq * (1.0 / _M)
    rs = lax.rsqrt(ex2 - mu * mu + _EPS)
    y = (z - mu) * rs
    return jnp.where(y > 0, y, 0.2 * y)


def _dense_body(st0, g0b, g1b, g2b, g3b, f4b,
                w04, w14, w24, w34, b04, b14, b24, b34, w4, b4,
                ob,
                sg1, sg2, sz5, st, st5, stage, sem):
    p = pl.program_id(0)
    i = pl.program_id(1)
    r = pl.ds(i * _BLK, _BLK)

    @pl.when((p == 0) & (i == 0))
    def _init():
        st[...] = jnp.zeros_like(st)
        st5[...] = jnp.zeros_like(st5)

    @pl.when(p == 0)
    def _phase0():
        for k, (gv, w, bb) in enumerate(
                ((g1b[...], w14, b14), (g2b[...], w24, b24),
                 (g3b[...], w34, b34))):
            if k == 0:
                sg1[r, :] = gv
            elif k == 1:
                sg2[r, :] = gv
            z = _mm(gv, w[...]) + bb[...]
            st[k:k + 1, :] += jnp.sum(z, axis=0, keepdims=True)
            st[k + 3:k + 4, :] += jnp.sum(z * z, axis=0, keepdims=True)

    @pl.when(p == 1)
    def _phase1():
        st0v = st0[...]
        stv = st[...]
        acc = f4b[...]
        branches = ((g0b[...], w04, b04, st0v[0:1, :], st0v[1:2, :]),
                    (sg1[r, :], w14, b14, stv[0:1, :], stv[3:4, :]),
                    (sg2[r, :], w24, b24, stv[1:2, :], stv[4:5, :]),
                    (g3b[...], w34, b34, stv[2:3, :], stv[5:6, :]))
        for gv, w, bb, sm, sq in branches:
            z = _mm(gv, w[...]) + bb[...]
            acc = acc + _norm_leaky(z, sm, sq)
        z = _mm(acc, w4[...]) + b4[...]
        sz5[r, :] = z
        st5[0:1, :] += jnp.sum(z, axis=0, keepdims=True)
        st5[1:2, :] += jnp.sum(z * z, axis=0, keepdims=True)

    @pl.when(p == 2)
    def _phase2():
        stv = st5[...]
        stage[...] = (_norm_leaky(sz5[r, :], stv[0:1, :], stv[1:2, :])
                      + f4b[...])
        pltpu.async_copy(stage, ob.at[r, :], sem).wait()


def _rows(c, blk=_BLK):
    return pl.BlockSpec((blk, c), lambda i: (i, 0))


def _rep(shape):
    return pl.BlockSpec(shape, lambda i: tuple(0 for _ in shape))


_ARB = pltpu.CompilerParams(dimension_semantics=("arbitrary",))


def kernel(num_point, f0, f1, f2, f3, f4, FPS_0, FPS_1, FPS_2, FPS_3,
           W04, b04, W14, b14, W24, b24, W34, b34, W4, b4):
    i04, o1, o2, o3 = _sc_chain()(FPS_0, FPS_1, FPS_2, FPS_3)
    g1, g2, g3 = _sc_gather()(
        o1, o2, o3,
        f1.reshape(_B * _N1, _C1),
        f2.reshape(_B * _N2, _C2), f3.reshape(_B * _N3, _C3))

    b04r, b14r = b04.reshape(1, _C4), b14.reshape(1, _C4)
    b24r, b34r = b24.reshape(1, _C4), b34.reshape(1, _C4)
    b4r = b4.reshape(1, _C4)
    f4f = f4.reshape(_M, _C4)
    f0t = jnp.transpose(f0, (0, 2, 1))
    arows = _ABLK * _S

    g0, st0 = pl.pallas_call(
        _a0_body,
        grid=(_B // _ABLK,),
        in_specs=[_rows(_S, _ABLK),
                  pl.BlockSpec((_ABLK, _C0, _N0), lambda i: (i, 0, 0)),
                  _rep((_C4, _C0)), _rep((1, _C4))],
        out_specs=(_rows(_C0, arows), _rep((2, _C4))),
        out_shape=(jax.ShapeDtypeStruct((_M, _C0), jnp.float32),
                   jax.ShapeDtypeStruct((2, _C4), jnp.float32)),
        compiler_params=_ARB,
    )(i04, f0t, W04, b04r)

    def _prow(c, cond):
        return pl.BlockSpec(
            (_BLK, c), lambda p, i: (jnp.where(cond(p), i, 0), 0))

    def _prep(shape):
        return pl.BlockSpec(shape, lambda p, i: tuple(0 for _ in shape))

    out = pl.pallas_call(
        _dense_body,
        grid=(3, _M // _BLK),
        in_specs=[_prep((2, _C4)),
                  _prow(_C0, lambda p: p == 1),
                  _prow(_C1, lambda p: p == 0),
                  _prow(_C2, lambda p: p == 0),
                  _prow(_C3, lambda p: p != 2),
                  _prow(_C4, lambda p: p != 0),
                  _prep((_C4, _C0)), _prep((_C4, _C1)),
                  _prep((_C4, _C2)), _prep((_C4, _C3)),
                  _prep((1, _C4)), _prep((1, _C4)),
                  _prep((1, _C4)), _prep((1, _C4)),
                  _prep((_C4, _C4)), _prep((1, _C4))],
        out_specs=pl.BlockSpec(memory_space=pl.ANY),
        out_shape=jax.ShapeDtypeStruct((_M, _C4), jnp.float32),
        scratch_shapes=[
            pltpu.VMEM((_M, _C1), jnp.float32),
            pltpu.VMEM((_M, _C2), jnp.float32),
            pltpu.VMEM((_M, _C4), jnp.float32),
            pltpu.VMEM((6, _C4), jnp.float32),
            pltpu.VMEM((2, _C4), jnp.float32),
            pltpu.VMEM((_BLK, _C4), jnp.float32),
            pltpu.SemaphoreType.DMA,
        ],
        compiler_params=pltpu.CompilerParams(
            dimension_semantics=("arbitrary", "arbitrary"),
            vmem_limit_bytes=120 * 1024 * 1024),
    )(st0, g0, g1, g2, g3, f4f, W04, W14, W24, W34,
      b04r, b14r, b24r, b34r, W4, b4r)

    return out.reshape(_B, _S, _C4)


# speedup vs baseline: 9.4893x; 1.0136x over previous
import functools

import jax
import jax.numpy as jnp
from jax import lax
from jax.experimental import pallas as pl
from jax.experimental.pallas import tpu as pltpu
from jax.experimental.pallas import tpu_sc as plsc

_B = 128
_S = 128
_N0, _N1, _N2, _N3 = 2048, 1024, 512, 256
_C0, _C1, _C2, _C3, _C4 = 64, 128, 128, 256, 256
_M = _B * _S
_EPS = 1e-5
_NC, _NS = 2, 16
_NW = _NC * _NS
_BPW = _B // _NW
_ABLK = 8
_BLK = 2048
_CH = 64

_SC_PARAMS = pltpu.CompilerParams(needs_layout_passes=False)


def _sc_mesh():
    return plsc.VectorSubcoreMesh(
        core_axis_name="c", subcore_axis_name="s",
        num_cores=_NC, num_subcores=_NS)


def _sc_chain():
    @functools.partial(
        pl.kernel,
        mesh=_sc_mesh(),
        compiler_params=_SC_PARAMS,
        out_type=(
            jax.ShapeDtypeStruct((_B, _S), jnp.int32),
            jax.ShapeDtypeStruct((_B, _S), jnp.int32),
            jax.ShapeDtypeStruct((_B, _S), jnp.int32),
            jax.ShapeDtypeStruct((_B, _S), jnp.int32),
        ),
        scratch_types=[
            pltpu.VMEM((_BPW, _N1), jnp.int32),
            pltpu.VMEM((_BPW, _N2), jnp.int32),
            pltpu.VMEM((_BPW, _N3), jnp.int32),
            pltpu.VMEM((_BPW, _S), jnp.int32),
            pltpu.VMEM((_BPW, _S), jnp.int32),
            pltpu.VMEM((_BPW, _S), jnp.int32),
            pltpu.VMEM((_BPW, _S), jnp.int32),
            pltpu.VMEM((_BPW, _S), jnp.int32),
        ],
    )
    def body(fps0, fps1, fps2, fps3,
             i04, o1, o2, o3,
             t0, t1, t2, t3, li0, gi1, gi2, gi3):
        wid = lax.axis_index("s") * _NC + lax.axis_index("c")
        b0 = wid * _BPW
        rows = pl.ds(b0, _BPW)
        pltpu.sync_copy(fps0.at[rows], t0)
        pltpu.sync_copy(fps1.at[rows], t1)
        pltpu.sync_copy(fps2.at[rows], t2)
        pltpu.sync_copy(fps3.at[rows], t3)

        for j in range(_BPW):
            jv = jnp.full((16,), j, jnp.int32)

            def chain(i, c, j=j, jv=jv):
                s = pl.ds(i * 16, 16)
                v3 = t3[j, s]
                v2 = plsc.load_gather(t2, [jv, v3])
                v1 = plsc.load_gather(t1, [jv, v2])
                v0 = plsc.load_gather(t0, [jv, v1])
                li0[j, s] = v0
                gi3[j, s] = v3 + (b0 + j) * _N3
                gi2[j, s] = v2 + (b0 + j) * _N2
                gi1[j, s] = v1 + (b0 + j) * _N1
                return c

            lax.fori_loop(0, _S // 16, chain, jnp.int32(0))

        pltpu.sync_copy(li0, i04.at[rows])
        pltpu.sync_copy(gi1, o1.at[rows])
        pltpu.sync_copy(gi2, o2.at[rows])
        pltpu.sync_copy(gi3, o3.at[rows])

    return body


def _sc_gather():
    @functools.partial(
        pl.kernel,
        mesh=_sc_mesh(),
        compiler_params=_SC_PARAMS,
        out_type=(
            jax.ShapeDtypeStruct((_M, _C1), jnp.float32),
            jax.ShapeDtypeStruct((_M, _C2), jnp.float32),
            jax.ShapeDtypeStruct((_M, _C3), jnp.float32),
        ),
        scratch_types=[
            pltpu.VMEM((_BPW, _S), jnp.int32),
            pltpu.VMEM((_BPW, _S), jnp.int32),
            pltpu.VMEM((_BPW, _S), jnp.int32),
            pltpu.VMEM((_CH, _C1), jnp.float32),
            pltpu.VMEM((_CH, _C2), jnp.float32),
            pltpu.VMEM((_CH, _C3), jnp.float32),
            pltpu.VMEM((_CH, _C1), jnp.float32),
            pltpu.VMEM((_CH, _C2), jnp.float32),
            pltpu.VMEM((_CH, _C3), jnp.float32),
            pltpu.SemaphoreType.DMA,
            pltpu.SemaphoreType.DMA,
            pltpu.SemaphoreType.DMA,
            pltpu.SemaphoreType.DMA,
        ],
    )
    def body(o1, o2, o3, f1, f2, f3,
             g1, g2, g3,
             i1b, i2b, i3b, r1a, r2a, r3a, r1b, r2b, r3b,
             gsa, gsb, wsa, wsb):
        wid = lax.axis_index("s") * _NC + lax.axis_index("c")
        b0 = wid * _BPW
        rows = pl.ds(b0, _BPW)
        pltpu.sync_copy(o1.at[rows], i1b)
        pltpu.sync_copy(o2.at[rows], i2b)
        pltpu.sync_copy(o3.at[rows], i3b)

        bufs = ((r1a, r2a, r3a), (r1b, r2b, r3b))
        gsems = (gsa, gsb)
        wsems = (wsa, wsb)
        nch = _BPW * _S // _CH

        def start_g(c):
            s = c % 2
            bj, h = divmod(c * _CH, _S)
            idx = (i1b.at[bj, pl.ds(h, _CH)], i2b.at[bj, pl.ds(h, _CH)],
                   i3b.at[bj, pl.ds(h, _CH)])
            return tuple(
                pltpu.async_copy(f.at[ix], buf, gsems[s])
                for f, ix, buf in zip((f1, f2, f3), idx, bufs[s]))

        def start_w(c):
            s = c % 2
            row = b0 * _S + c * _CH
            return tuple(
                pltpu.async_copy(buf, g.at[pl.ds(row, _CH)], wsems[s])
                for g, buf in zip((g1, g2, g3), bufs[s]))

        pg = {0: start_g(0)}
        pw = {}
        for c in range(1, nch + 1):
            if c <= nch - 1:
                if c >= 2:
                    for w in pw.pop(c - 2):
                        w.wait()
                pg[c] = start_g(c)
            for g in pg.pop(c - 1):
                g.wait()
            pw[c - 1] = start_w(c - 1)
        for w in pw[nch - 2]:
            w.wait()
        for w in pw[nch - 1]:
            w.wait()

    return body


def _mm(x, w):
    return lax.dot_general(x, w, (((1,), (1,)), ((), ())),
                           preferred_element_type=jnp.float32)


def _a0_body(i04b, f0tb, w04, b04, g0o, st0):
    i = pl.program_id(0)

    @pl.when(i == 0)
    def _init():
        st0[...] = jnp.zeros_like(st0)

    idx = i04b[...]
    f0t = f0tb[...].astype(jnp.bfloat16)
    parts = []
    for j in range(_ABLK):
        row = idx[j:j + 1, :]
        n_iota = lax.broadcasted_iota(jnp.int32, (_N0, _S), 0)
        oh = (n_iota == row).astype(jnp.bfloat16)
        parts.append(lax.dot_general(
            oh, f0t[j], (((0,), (1,)), ((), ())),
            preferred_element_type=jnp.float32))
    g0 = jnp.concatenate(parts, axis=0)
    g0o[...] = g0
    z = _mm(g0, w04[...]) + b04[...]
    st0[0:1, :] += jnp.sum(z, axis=0, keepdims=True)
    st0[1:2, :] += jnp.sum(z * z, axis=0, keepdims=True)


def _norm_leaky(z, sm, sq):
    mu = sm * (1.0 / _M)
    ex2 = sq * (1.0 / _M)
    rs = lax.rsqrt(ex2 - mu * mu + _EPS)
    y = (z - mu) * rs
    return jnp.where(y > 0, y, 0.2 * y)


def _dense_body(st0, g0b, g1b, g2b, g3b, f4b,
                w04, w14, w24, w34, b04, b14, b24, b34, w4, b4,
                ob,
                sg1, sg2, sg3, sf4, sz5, st, st5, stage, sem):
    p = pl.program_id(0)
    i = pl.program_id(1)
    r = pl.ds(i * _BLK, _BLK)

    @pl.when((p == 0) & (i == 0))
    def _init():
        st[...] = jnp.zeros_like(st)
        st5[...] = jnp.zeros_like(st5)

    @pl.when(p == 0)
    def _phase0():
        for k, (gv, sg, w, bb) in enumerate(
                ((g1b[...], sg1, w14, b14), (g2b[...], sg2, w24, b24),
                 (g3b[...], sg3, w34, b34))):
            sg[r, :] = gv.astype(jnp.bfloat16)
            z = _mm(gv, w[...]) + bb[...]
            st[k:k + 1, :] += jnp.sum(z, axis=0, keepdims=True)
            st[k + 3:k + 4, :] += jnp.sum(z * z, axis=0, keepdims=True)

    @pl.when(p == 1)
    def _phase1():
        st0v = st0[...]
        stv = st[...]
        f4v = f4b[...]
        sf4[r, :] = f4v.astype(jnp.bfloat16)
        acc = f4v
        branches = ((g0b[...], w04, b04, st0v[0:1, :], st0v[1:2, :]),
                    (sg1[r, :].astype(jnp.float32), w14, b14,
                     stv[0:1, :], stv[3:4, :]),
                    (sg2[r, :].astype(jnp.float32), w24, b24,
                     stv[1:2, :], stv[4:5, :]),
                    (sg3[r, :].astype(jnp.float32), w34, b34,
                     stv[2:3, :], stv[5:6, :]))
        for gv, w, bb, sm, sq in branches:
            z = _mm(gv, w[...]) + bb[...]
            acc = acc + _norm_leaky(z, sm, sq)
        z = _mm(acc, w4[...]) + b4[...]
        sz5[r, :] = z
        st5[0:1, :] += jnp.sum(z, axis=0, keepdims=True)
        st5[1:2, :] += jnp.sum(z * z, axis=0, keepdims=True)

    @pl.when(p == 2)
    def _phase2():
        stv = st5[...]
        stage[...] = (_norm_leaky(sz5[r, :], stv[0:1, :], stv[1:2, :])
                      + sf4[r, :].astype(jnp.float32))
        pltpu.async_copy(stage, ob.at[r, :], sem).wait()


def _rows(c, blk=_BLK):
    return pl.BlockSpec((blk, c), lambda i: (i, 0))


def _rep(shape):
    return pl.BlockSpec(shape, lambda i: tuple(0 for _ in shape))


_ARB = pltpu.CompilerParams(dimension_semantics=("arbitrary",))


def kernel(num_point, f0, f1, f2, f3, f4, FPS_0, FPS_1, FPS_2, FPS_3,
           W04, b04, W14, b14, W24, b24, W34, b34, W4, b4):
    i04, o1, o2, o3 = _sc_chain()(FPS_0, FPS_1, FPS_2, FPS_3)
    g1, g2, g3 = _sc_gather()(
        o1, o2, o3,
        f1.reshape(_B * _N1, _C1),
        f2.reshape(_B * _N2, _C2), f3.reshape(_B * _N3, _C3))

    b04r, b14r = b04.reshape(1, _C4), b14.reshape(1, _C4)
    b24r, b34r = b24.reshape(1, _C4), b34.reshape(1, _C4)
    b4r = b4.reshape(1, _C4)
    f4f = f4.reshape(_M, _C4)
    f0t = jnp.transpose(f0, (0, 2, 1))
    arows = _ABLK * _S

    g0, st0 = pl.pallas_call(
        _a0_body,
        grid=(_B // _ABLK,),
        in_specs=[_rows(_S, _ABLK),
                  pl.BlockSpec((_ABLK, _C0, _N0), lambda i: (i, 0, 0)),
                  _rep((_C4, _C0)), _rep((1, _C4))],
        out_specs=(_rows(_C0, arows), _rep((2, _C4))),
        out_shape=(jax.ShapeDtypeStruct((_M, _C0), jnp.float32),
                   jax.ShapeDtypeStruct((2, _C4), jnp.float32)),
        compiler_params=_ARB,
    )(i04, f0t, W04, b04r)

    def _prow(c, cond):
        return pl.BlockSpec(
            (_BLK, c), lambda p, i: (jnp.where(cond(p), i, 0), 0))

    def _prep(shape):
        return pl.BlockSpec(shape, lambda p, i: tuple(0 for _ in shape))

    out = pl.pallas_call(
        _dense_body,
        grid=(3, _M // _BLK),
        in_specs=[_prep((2, _C4)),
                  _prow(_C0, lambda p: p == 1),
                  _prow(_C1, lambda p: p == 0),
                  _prow(_C2, lambda p: p == 0),
                  _prow(_C3, lambda p: p == 0),
                  _prow(_C4, lambda p: p == 1),
                  _prep((_C4, _C0)), _prep((_C4, _C1)),
                  _prep((_C4, _C2)), _prep((_C4, _C3)),
                  _prep((1, _C4)), _prep((1, _C4)),
                  _prep((1, _C4)), _prep((1, _C4)),
                  _prep((_C4, _C4)), _prep((1, _C4))],
        out_specs=pl.BlockSpec(memory_space=pl.ANY),
        out_shape=jax.ShapeDtypeStruct((_M, _C4), jnp.float32),
        scratch_shapes=[
            pltpu.VMEM((_M, _C1), jnp.bfloat16),
            pltpu.VMEM((_M, _C2), jnp.bfloat16),
            pltpu.VMEM((_M, _C3), jnp.bfloat16),
            pltpu.VMEM((_M, _C4), jnp.bfloat16),
            pltpu.VMEM((_M, _C4), jnp.float32),
            pltpu.VMEM((6, _C4), jnp.float32),
            pltpu.VMEM((2, _C4), jnp.float32),
            pltpu.VMEM((_BLK, _C4), jnp.float32),
            pltpu.SemaphoreType.DMA,
        ],
        compiler_params=pltpu.CompilerParams(
            dimension_semantics=("arbitrary", "arbitrary"),
            vmem_limit_bytes=120 * 1024 * 1024),
    )(st0, g0, g1, g2, g3, f4f, W04, W14, W24, W34,
      b04r, b14r, b24r, b34r, W4, b4r)

    return out.reshape(_B, _S, _C4)
